# baseline (speedup 1.0000x reference)
import functools

import jax
import jax.numpy as jnp
from jax import lax
from jax.experimental import pallas as pl
from jax.experimental.pallas import tpu as pltpu
from jax.experimental.pallas import tpu_sc as plsc

N = 10000
E = 320000
F = 128
H = 128
C = 64

NC = 2
NS = 16
NW = NC * NS
L = 16

CH = 64
NSLOT = 4
P = 7
N1 = 10752
NP = N1 // P
RPP = NP // NS
NCH0 = 176
NCH1 = 144
E_PAD = NS * (NCH0 + NCH1) * CH
CNT_W = 16
IGN = -1

_mesh = plsc.VectorSubcoreMesh(core_axis_name="c", subcore_axis_name="s")


def _agg_body(with_counts, x_hbm, src_hbm, dst_hbm, zrow_hbm, zcnt_hbm,
              ones_hbm, *refs):
    if with_counts:
        sum_out, cnt_out = refs[0], refs[1]
        refs = refs[2:]
    else:
        sum_out = refs[0]
        refs = refs[1:]
    srcs, dsts = refs[0:4], refs[4:8]
    msrcs, mdsts = refs[8:12], refs[12:16]
    rowss = refs[16:20]
    refs = refs[20:]
    if with_counts:
        ones_v, cbuf_v, sum_sh, cnt_sh = refs[0:4]
        sems = refs[4:]
    else:
        sum_sh = refs[0]
        sems = refs[1:]
    gsems, ssems, isems = sems[0:4], sems[4:8], sems[8:12]
    c = lax.axis_index("c")
    s = lax.axis_index("s")
    t0 = NCH0 * CH
    t1 = NCH1 * CH
    e0 = jnp.where(c == 0, s * t0, NS * t0 + s * t1)
    nch = jnp.where(c == 0, NCH0, NCH1)
    r0 = s * RPP

    if with_counts:
        pltpu.sync_copy(ones_hbm, ones_v)

    def gather_desc(u):
        return pltpu.make_async_copy(
            x_hbm.at[plsc.Indices(msrcs[u], ignored_value=IGN)],
            rowss[u], gsems[u])

    def scat_desc(u):
        return pltpu.make_async_copy(
            rowss[u], sum_sh.at[plsc.Indices(mdsts[u], ignored_value=IGN)],
            ssems[u])

    def cnt_desc(u):
        return pltpu.make_async_copy(
            ones_v, cnt_sh.at[plsc.Indices(mdsts[u], ignored_value=IGN)],
            ssems[u])

    for p in range(P):
        lo = p * NP

        pltpu.sync_copy(zrow_hbm, rowss[0])
        pltpu.sync_copy(rowss[0], sum_sh.at[pl.ds(r0, CH)])
        pltpu.sync_copy(rowss[0].at[pl.ds(0, RPP - CH)],
                        sum_sh.at[pl.ds(r0 + CH, RPP - CH)])
        if with_counts:
            pltpu.sync_copy(zcnt_hbm, cbuf_v)
            pltpu.sync_copy(cbuf_v, cnt_sh.at[pl.ds(r0, RPP)])
        plsc.subcore_barrier()

        def iload(cj, u):
            off = e0 + cj * CH
            pltpu.async_copy(src_hbm.at[pl.ds(off, CH)], srcs[u], isems[u])
            pltpu.async_copy(dst_hbm.at[pl.ds(off, CH)], dsts[u], isems[u])

        def iwait(cj, u):
            off = e0 + cj * CH
            pltpu.make_async_copy(
                src_hbm.at[pl.ds(off, CH)], srcs[u], isems[u]).wait()
            pltpu.make_async_copy(
                dst_hbm.at[pl.ds(off, CH)], dsts[u], isems[u]).wait()

        def mask_gather(cj, u):
            for k in range(CH // L):
                sl = pl.ds(k * L, L)
                d = dsts[u][sl]
                sv = srcs[u][sl]
                inr = (d >= lo) & (d < lo + NP)
                mdsts[u][sl] = jnp.where(inr, d - lo, IGN)
                msrcs[u][sl] = jnp.where(inr, sv, IGN)
            gather_desc(u).start()

        iload(0, 0)
        iload(1, 1)
        iwait(0, 0)
        mask_gather(0, 0)
        iwait(1, 1)
        mask_gather(1, 1)
        iload(2, 2)
        iload(3, 3)

        def body(jj, carry):
            for u in range(NSLOT):
                cj = NSLOT * jj + u
                u2 = (u + 2) % NSLOT

                @pl.when(cj >= 2)
                def _():
                    scat_desc(u2).wait()
                    if with_counts:
                        cnt_desc(u2).wait()

                @pl.when(cj + 2 < nch)
                def _():
                    iwait(cj + 2, u2)
                    mask_gather(cj + 2, u2)

                @pl.when(cj + 4 < nch)
                def _():
                    iload(cj + 4, u)

                gather_desc(u).wait()
                scat_desc(u).start(add=True)
                if with_counts:
                    cnt_desc(u).start(add=True)
            return carry

        lax.fori_loop(0, nch // NSLOT, body, 0)
        for u in (2, 3):
            scat_desc(u).wait()
            if with_counts:
                cnt_desc(u).wait()

        plsc.subcore_barrier()
        pltpu.sync_copy(sum_sh.at[pl.ds(r0, CH)], rowss[0])
        pltpu.sync_copy(rowss[0], sum_out.at[c, pl.ds(lo + r0, CH)])
        pltpu.sync_copy(sum_sh.at[pl.ds(r0 + CH, RPP - CH)],
                        rowss[1].at[pl.ds(0, RPP - CH)])
        pltpu.sync_copy(rowss[1].at[pl.ds(0, RPP - CH)],
                        sum_out.at[c, pl.ds(lo + r0 + CH, RPP - CH)])
        if with_counts:
            pltpu.sync_copy(cnt_sh.at[pl.ds(r0, RPP)], cbuf_v)
            pltpu.sync_copy(cbuf_v, cnt_out.at[c, pl.ds(lo + r0, RPP)])


_agg_with_counts = functools.partial(
    pl.kernel,
    mesh=_mesh,
    out_type=[
        jax.ShapeDtypeStruct((NC, N1, F), jnp.float32),
        jax.ShapeDtypeStruct((NC, N1, CNT_W), jnp.float32),
    ],
    scratch_types=(
        [pltpu.VMEM((CH,), jnp.int32)] * 16
        + [pltpu.VMEM((CH, F), jnp.float32)] * 4
        + [
            pltpu.VMEM((CH, CNT_W), jnp.float32),
            pltpu.VMEM((RPP, CNT_W), jnp.float32),
            pltpu.VMEM_SHARED((NP, F), jnp.float32),
            pltpu.VMEM_SHARED((NP, CNT_W), jnp.float32),
        ]
        + [pltpu.SemaphoreType.DMA] * 12
    ),
)(functools.partial(_agg_body, True))

_agg_no_counts = functools.partial(
    pl.kernel,
    mesh=_mesh,
    out_type=jax.ShapeDtypeStruct((NC, N1, F), jnp.float32),
    scratch_types=(
        [pltpu.VMEM((CH,), jnp.int32)] * 16
        + [pltpu.VMEM((CH, F), jnp.float32)] * 4
        + [pltpu.VMEM_SHARED((NP, F), jnp.float32)]
        + [pltpu.SemaphoreType.DMA] * 12
    ),
)(functools.partial(_agg_body, False))


BLK = 2000


def _sage1_body(sump_ref, cntp_ref, x_ref, w1l_ref, b1_ref, w1r_ref, h1_ref):
    s = sump_ref[0] + sump_ref[1]
    cnt = cntp_ref[0, :, 0:1] + cntp_ref[1, :, 0:1]
    mean = s / jnp.maximum(cnt, 1.0)
    h1_ref[...] = (
        jnp.dot(mean, w1l_ref[...], preferred_element_type=jnp.float32)
        + b1_ref[...]
        + jnp.dot(x_ref[...], w1r_ref[...], preferred_element_type=jnp.float32)
    )


def _sage2_fc_body(sump_ref, cntp_ref, h1_ref, w2l_ref, b2_ref, w2r_ref,
                   fcw1_ref, fcb1_ref, fcw2_ref, fcb2_ref, out_ref):
    s = sump_ref[0] + sump_ref[1]
    cnt = cntp_ref[0, :, 0:1] + cntp_ref[1, :, 0:1]
    mean = s / jnp.maximum(cnt, 1.0)
    h2 = (
        jnp.dot(mean, w2l_ref[...], preferred_element_type=jnp.float32)
        + b2_ref[...]
        + jnp.dot(h1_ref[...], w2r_ref[...], preferred_element_type=jnp.float32)
    )
    t = jnp.dot(h2, fcw1_ref[...], preferred_element_type=jnp.float32) + fcb1_ref[...]
    out_ref[...] = (
        jnp.dot(t, fcw2_ref[...], preferred_element_type=jnp.float32) + fcb2_ref[...]
    )


def _full(shape):
    return pl.BlockSpec(shape, lambda i: tuple(0 for _ in shape))


def _rows(shape):
    return pl.BlockSpec(shape, lambda i: (i,) + tuple(0 for _ in shape[1:]))


def _rows3(shape):
    return pl.BlockSpec(shape, lambda i: (0, i, 0))


def kernel(x, edge_idx, edge_weight, W1l, b1, W1r, W2l, b2, W2r,
           fcW1, fcb1, fcW2, fcb2):
    src = edge_idx[0]
    dst = edge_idx[1]
    pad = E_PAD - E
    src_p = jnp.concatenate([src, jnp.zeros((pad,), jnp.int32)])
    dst_p = jnp.concatenate([dst, jnp.full((pad,), N, jnp.int32)])
    zrow = jnp.zeros((CH, F), jnp.float32)
    zcnt = jnp.zeros((RPP, CNT_W), jnp.float32)
    ones = jnp.ones((CH, CNT_W), jnp.float32)

    sum1_p, cnt_p = _agg_with_counts(x, src_p, dst_p, zrow, zcnt, ones)

    h1 = pl.pallas_call(
        _sage1_body,
        grid=(N // BLK,),
        in_specs=[
            _rows3((NC, BLK, F)),
            _rows3((NC, BLK, CNT_W)),
            _rows((BLK, F)),
            _full((F, H)),
            _full((1, H)),
            _full((F, H)),
        ],
        out_specs=_rows((BLK, H)),
        out_shape=jax.ShapeDtypeStruct((N, H), jnp.float32),
    )(sum1_p, cnt_p, x, W1l, b1.reshape(1, H), W1r)

    sum2_p = _agg_no_counts(h1, src_p, dst_p, zrow, zcnt, ones)

    out = pl.pallas_call(
        _sage2_fc_body,
        grid=(N // BLK,),
        in_specs=[
            _rows3((NC, BLK, H)),
            _rows3((NC, BLK, CNT_W)),
            _rows((BLK, H)),
            _full((H, H)),
            _full((1, H)),
            _full((H, H)),
            _full((H, C)),
            _full((1, C)),
            _full((C, 1)),
            _full((1, 1)),
        ],
        out_specs=_rows((BLK, 1)),
        out_shape=jax.ShapeDtypeStruct((N, 1), jnp.float32),
    )(sum2_p, cnt_p, h1, W2l, b2.reshape(1, H), W2r,
      fcW1, fcb1.reshape(1, C), fcW2, fcb2.reshape(1, 1))

    return out


# speedup vs baseline: 1.1040x; 1.1040x over previous
import functools

import jax
import jax.numpy as jnp
from jax import lax
from jax.experimental import pallas as pl
from jax.experimental.pallas import tpu as pltpu
from jax.experimental.pallas import tpu_sc as plsc

N = 10000
E = 320000
F = 128
H = 128
C = 64

NC = 2
NS = 16
NW = NC * NS
L = 16

CH = 64
NSLOT = 4
P = 7
N1 = 10752
NP = N1 // P
RPP = NP // NS
NCH0 = 208
NCH1 = 112
E_PAD = NS * (NCH0 + NCH1) * CH
CNT_W = 16
IGN = -1

_mesh = plsc.VectorSubcoreMesh(core_axis_name="c", subcore_axis_name="s")


def _agg_body(with_counts, x_hbm, src_hbm, dst_hbm, zrow_hbm, zcnt_hbm,
              ones_hbm, *refs):
    if with_counts:
        sum_out, cnt_out = refs[0], refs[1]
        refs = refs[2:]
    else:
        sum_out = refs[0]
        refs = refs[1:]
    srcs, dsts = refs[0:4], refs[4:8]
    msrcs, mdsts = refs[8:12], refs[12:16]
    rowss = refs[16:20]
    refs = refs[20:]
    if with_counts:
        ones_v, cbuf_v, sum_sh, cnt_sh = refs[0:4]
        sems = refs[4:]
    else:
        sum_sh = refs[0]
        sems = refs[1:]
    gsems, ssems, isems = sems[0:4], sems[4:8], sems[8:12]
    c = lax.axis_index("c")
    s = lax.axis_index("s")
    t0 = NCH0 * CH
    t1 = NCH1 * CH
    e0 = jnp.where(c == 0, s * t0, NS * t0 + s * t1)
    nch = jnp.where(c == 0, NCH0, NCH1)
    r0 = s * RPP

    if with_counts:
        pltpu.sync_copy(ones_hbm, ones_v)

    def gather_desc(u):
        return pltpu.make_async_copy(
            x_hbm.at[plsc.Indices(msrcs[u], ignored_value=IGN)],
            rowss[u], gsems[u])

    def scat_desc(u):
        return pltpu.make_async_copy(
            rowss[u], sum_sh.at[plsc.Indices(mdsts[u], ignored_value=IGN)],
            ssems[u])

    def cnt_desc(u):
        return pltpu.make_async_copy(
            ones_v, cnt_sh.at[plsc.Indices(mdsts[u], ignored_value=IGN)],
            ssems[u])

    for p in range(P):
        lo = p * NP

        pltpu.sync_copy(zrow_hbm, rowss[0])
        pltpu.sync_copy(rowss[0], sum_sh.at[pl.ds(r0, CH)])
        pltpu.sync_copy(rowss[0].at[pl.ds(0, RPP - CH)],
                        sum_sh.at[pl.ds(r0 + CH, RPP - CH)])
        if with_counts:
            pltpu.sync_copy(zcnt_hbm, cbuf_v)
            pltpu.sync_copy(cbuf_v, cnt_sh.at[pl.ds(r0, RPP)])
        plsc.subcore_barrier()

        def iload(cj, u):
            off = e0 + cj * CH
            pltpu.async_copy(src_hbm.at[pl.ds(off, CH)], srcs[u], isems[u])
            pltpu.async_copy(dst_hbm.at[pl.ds(off, CH)], dsts[u], isems[u])

        def iwait(cj, u):
            off = e0 + cj * CH
            pltpu.make_async_copy(
                src_hbm.at[pl.ds(off, CH)], srcs[u], isems[u]).wait()
            pltpu.make_async_copy(
                dst_hbm.at[pl.ds(off, CH)], dsts[u], isems[u]).wait()

        def mask_gather(cj, u):
            for k in range(CH // L):
                sl = pl.ds(k * L, L)
                d = dsts[u][sl]
                sv = srcs[u][sl]
                inr = (d >= lo) & (d < lo + NP)
                mdsts[u][sl] = jnp.where(inr, d - lo, IGN)
                msrcs[u][sl] = jnp.where(inr, sv, IGN)
            gather_desc(u).start()

        iload(0, 0)
        iload(1, 1)
        iwait(0, 0)
        mask_gather(0, 0)
        iwait(1, 1)
        mask_gather(1, 1)
        iload(2, 2)
        iload(3, 3)

        def body(jj, carry):
            for u in range(NSLOT):
                cj = NSLOT * jj + u
                u2 = (u + 2) % NSLOT

                @pl.when(cj >= 2)
                def _():
                    scat_desc(u2).wait()
                    if with_counts:
                        cnt_desc(u2).wait()

                @pl.when(cj + 2 < nch)
                def _():
                    iwait(cj + 2, u2)
                    mask_gather(cj + 2, u2)

                @pl.when(cj + 4 < nch)
                def _():
                    iload(cj + 4, u)

                gather_desc(u).wait()
                scat_desc(u).start(add=True)
                if with_counts:
                    cnt_desc(u).start(add=True)
            return carry

        lax.fori_loop(0, nch // NSLOT, body, 0)
        for u in (2, 3):
            scat_desc(u).wait()
            if with_counts:
                cnt_desc(u).wait()

        plsc.subcore_barrier()
        pltpu.sync_copy(sum_sh.at[pl.ds(r0, CH)], rowss[0])
        pltpu.sync_copy(rowss[0], sum_out.at[c, pl.ds(lo + r0, CH)])
        pltpu.sync_copy(sum_sh.at[pl.ds(r0 + CH, RPP - CH)],
                        rowss[1].at[pl.ds(0, RPP - CH)])
        pltpu.sync_copy(rowss[1].at[pl.ds(0, RPP - CH)],
                        sum_out.at[c, pl.ds(lo + r0 + CH, RPP - CH)])
        if with_counts:
            pltpu.sync_copy(cnt_sh.at[pl.ds(r0, RPP)], cbuf_v)
            pltpu.sync_copy(cbuf_v, cnt_out.at[c, pl.ds(lo + r0, RPP)])


_agg_with_counts = functools.partial(
    pl.kernel,
    mesh=_mesh,
    out_type=[
        jax.ShapeDtypeStruct((NC, N1, F), jnp.float32),
        jax.ShapeDtypeStruct((NC, N1, CNT_W), jnp.float32),
    ],
    scratch_types=(
        [pltpu.VMEM((CH,), jnp.int32)] * 16
        + [pltpu.VMEM((CH, F), jnp.float32)] * 4
        + [
            pltpu.VMEM((CH, CNT_W), jnp.float32),
            pltpu.VMEM((RPP, CNT_W), jnp.float32),
            pltpu.VMEM_SHARED((NP, F), jnp.float32),
            pltpu.VMEM_SHARED((NP, CNT_W), jnp.float32),
        ]
        + [pltpu.SemaphoreType.DMA] * 12
    ),
)(functools.partial(_agg_body, True))

_agg_no_counts = functools.partial(
    pl.kernel,
    mesh=_mesh,
    out_type=jax.ShapeDtypeStruct((NC, N1, F), jnp.float32),
    scratch_types=(
        [pltpu.VMEM((CH,), jnp.int32)] * 16
        + [pltpu.VMEM((CH, F), jnp.float32)] * 4
        + [pltpu.VMEM_SHARED((NP, F), jnp.float32)]
        + [pltpu.SemaphoreType.DMA] * 12
    ),
)(functools.partial(_agg_body, False))


BLK = 2000


def _sage1_body(sump_ref, cntp_ref, x_ref, w1l_ref, b1_ref, w1r_ref, h1_ref):
    s = sump_ref[0] + sump_ref[1]
    cnt = cntp_ref[0, :, 0:1] + cntp_ref[1, :, 0:1]
    mean = s / jnp.maximum(cnt, 1.0)
    h1_ref[...] = (
        jnp.dot(mean, w1l_ref[...], preferred_element_type=jnp.float32)
        + b1_ref[...]
        + jnp.dot(x_ref[...], w1r_ref[...], preferred_element_type=jnp.float32)
    )


def _sage2_fc_body(sump_ref, cntp_ref, h1_ref, w2l_ref, b2_ref, w2r_ref,
                   fcw1_ref, fcb1_ref, fcw2_ref, fcb2_ref, out_ref):
    s = sump_ref[0] + sump_ref[1]
    cnt = cntp_ref[0, :, 0:1] + cntp_ref[1, :, 0:1]
    mean = s / jnp.maximum(cnt, 1.0)
    h2 = (
        jnp.dot(mean, w2l_ref[...], preferred_element_type=jnp.float32)
        + b2_ref[...]
        + jnp.dot(h1_ref[...], w2r_ref[...], preferred_element_type=jnp.float32)
    )
    t = jnp.dot(h2, fcw1_ref[...], preferred_element_type=jnp.float32) + fcb1_ref[...]
    out_ref[...] = (
        jnp.dot(t, fcw2_ref[...], preferred_element_type=jnp.float32) + fcb2_ref[...]
    )


def _full(shape):
    return pl.BlockSpec(shape, lambda i: tuple(0 for _ in shape))


def _rows(shape):
    return pl.BlockSpec(shape, lambda i: (i,) + tuple(0 for _ in shape[1:]))


def _rows3(shape):
    return pl.BlockSpec(shape, lambda i: (0, i, 0))


def kernel(x, edge_idx, edge_weight, W1l, b1, W1r, W2l, b2, W2r,
           fcW1, fcb1, fcW2, fcb2):
    src = edge_idx[0]
    dst = edge_idx[1]
    pad = E_PAD - E
    src_p = jnp.concatenate([src, jnp.zeros((pad,), jnp.int32)])
    dst_p = jnp.concatenate([dst, jnp.full((pad,), N, jnp.int32)])
    zrow = jnp.zeros((CH, F), jnp.float32)
    zcnt = jnp.zeros((RPP, CNT_W), jnp.float32)
    ones = jnp.ones((CH, CNT_W), jnp.float32)

    sum1_p, cnt_p = _agg_with_counts(x, src_p, dst_p, zrow, zcnt, ones)

    h1 = pl.pallas_call(
        _sage1_body,
        grid=(N // BLK,),
        in_specs=[
            _rows3((NC, BLK, F)),
            _rows3((NC, BLK, CNT_W)),
            _rows((BLK, F)),
            _full((F, H)),
            _full((1, H)),
            _full((F, H)),
        ],
        out_specs=_rows((BLK, H)),
        out_shape=jax.ShapeDtypeStruct((N, H), jnp.float32),
    )(sum1_p, cnt_p, x, W1l, b1.reshape(1, H), W1r)

    sum2_p = _agg_no_counts(h1, src_p, dst_p, zrow, zcnt, ones)

    out = pl.pallas_call(
        _sage2_fc_body,
        grid=(N // BLK,),
        in_specs=[
            _rows3((NC, BLK, H)),
            _rows3((NC, BLK, CNT_W)),
            _rows((BLK, H)),
            _full((H, H)),
            _full((1, H)),
            _full((H, H)),
            _full((H, C)),
            _full((1, C)),
            _full((C, 1)),
            _full((1, 1)),
        ],
        out_specs=_rows((BLK, 1)),
        out_shape=jax.ShapeDtypeStruct((N, 1), jnp.float32),
    )(sum2_p, cnt_p, h1, W2l, b2.reshape(1, H), W2r,
      fcW1, fcb1.reshape(1, C), fcW2, fcb2.reshape(1, 1))

    return out


# speedup vs baseline: 1.1615x; 1.0521x over previous
import functools

import jax
import jax.numpy as jnp
from jax import lax
from jax.experimental import pallas as pl
from jax.experimental.pallas import tpu as pltpu
from jax.experimental.pallas import tpu_sc as plsc

N = 10000
E = 320000
F = 128
H = 128
C = 64

NC = 2
NS = 16
NW = NC * NS
L = 16

CH = 64
NSLOT = 4
P = 7
N1 = 10752
NP = N1 // P
RPP = NP // NS
NCH0 = 224
NCH1 = 96
E_PAD = NS * (NCH0 + NCH1) * CH
CNT_W = 16
IGN = -1

_mesh = plsc.VectorSubcoreMesh(core_axis_name="c", subcore_axis_name="s")


def _agg_body(with_counts, x_hbm, src_hbm, dst_hbm, zrow_hbm, zcnt_hbm,
              ones_hbm, *refs):
    if with_counts:
        sum_out, cnt_out = refs[0], refs[1]
        refs = refs[2:]
    else:
        sum_out = refs[0]
        refs = refs[1:]
    srcs, dsts = refs[0:4], refs[4:8]
    msrcs, mdsts = refs[8:12], refs[12:16]
    rowss = refs[16:20]
    refs = refs[20:]
    if with_counts:
        ones_v, cbuf_v, sum_sh, cnt_sh = refs[0:4]
        sems = refs[4:]
    else:
        sum_sh = refs[0]
        sems = refs[1:]
    gsems, ssems, isems = sems[0:4], sems[4:8], sems[8:12]
    c = lax.axis_index("c")
    s = lax.axis_index("s")
    t0 = NCH0 * CH
    t1 = NCH1 * CH
    e0 = jnp.where(c == 0, s * t0, NS * t0 + s * t1)
    nch = jnp.where(c == 0, NCH0, NCH1)
    r0 = s * RPP

    if with_counts:
        pltpu.sync_copy(ones_hbm, ones_v)

    def gather_desc(u):
        return pltpu.make_async_copy(
            x_hbm.at[plsc.Indices(msrcs[u], ignored_value=IGN)],
            rowss[u], gsems[u])

    def scat_desc(u):
        return pltpu.make_async_copy(
            rowss[u], sum_sh.at[plsc.Indices(mdsts[u], ignored_value=IGN)],
            ssems[u])

    def cnt_desc(u):
        return pltpu.make_async_copy(
            ones_v, cnt_sh.at[plsc.Indices(mdsts[u], ignored_value=IGN)],
            ssems[u])

    for p in range(P):
        lo = p * NP

        pltpu.sync_copy(zrow_hbm, rowss[0])
        pltpu.sync_copy(rowss[0], sum_sh.at[pl.ds(r0, CH)])
        pltpu.sync_copy(rowss[0].at[pl.ds(0, RPP - CH)],
                        sum_sh.at[pl.ds(r0 + CH, RPP - CH)])
        if with_counts:
            pltpu.sync_copy(zcnt_hbm, cbuf_v)
            pltpu.sync_copy(cbuf_v, cnt_sh.at[pl.ds(r0, RPP)])
        plsc.subcore_barrier()

        def iload(cj, u):
            off = e0 + cj * CH
            pltpu.async_copy(src_hbm.at[pl.ds(off, CH)], srcs[u], isems[u])
            pltpu.async_copy(dst_hbm.at[pl.ds(off, CH)], dsts[u], isems[u])

        def iwait(cj, u):
            off = e0 + cj * CH
            pltpu.make_async_copy(
                src_hbm.at[pl.ds(off, CH)], srcs[u], isems[u]).wait()
            pltpu.make_async_copy(
                dst_hbm.at[pl.ds(off, CH)], dsts[u], isems[u]).wait()

        def mask_gather(cj, u):
            for k in range(CH // L):
                sl = pl.ds(k * L, L)
                d = dsts[u][sl]
                sv = srcs[u][sl]
                inr = (d >= lo) & (d < lo + NP)
                mdsts[u][sl] = jnp.where(inr, d - lo, IGN)
                msrcs[u][sl] = jnp.where(inr, sv, IGN)
            gather_desc(u).start()

        iload(0, 0)
        iload(1, 1)
        iwait(0, 0)
        mask_gather(0, 0)
        iwait(1, 1)
        mask_gather(1, 1)
        iload(2, 2)
        iload(3, 3)

        def body(jj, carry):
            for u in range(NSLOT):
                cj = NSLOT * jj + u
                u2 = (u + 2) % NSLOT

                @pl.when(cj >= 2)
                def _():
                    scat_desc(u2).wait()
                    if with_counts:
                        cnt_desc(u2).wait()

                @pl.when(cj + 2 < nch)
                def _():
                    iwait(cj + 2, u2)
                    mask_gather(cj + 2, u2)

                @pl.when(cj + 4 < nch)
                def _():
                    iload(cj + 4, u)

                gather_desc(u).wait()
                scat_desc(u).start(add=True)
                if with_counts:
                    cnt_desc(u).start(add=True)
            return carry

        lax.fori_loop(0, nch // NSLOT, body, 0)
        for u in (2, 3):
            scat_desc(u).wait()
            if with_counts:
                cnt_desc(u).wait()

        plsc.subcore_barrier()
        pltpu.sync_copy(sum_sh.at[pl.ds(r0, CH)], rowss[0])
        pltpu.sync_copy(rowss[0], sum_out.at[c, pl.ds(lo + r0, CH)])
        pltpu.sync_copy(sum_sh.at[pl.ds(r0 + CH, RPP - CH)],
                        rowss[1].at[pl.ds(0, RPP - CH)])
        pltpu.sync_copy(rowss[1].at[pl.ds(0, RPP - CH)],
                        sum_out.at[c, pl.ds(lo + r0 + CH, RPP - CH)])
        if with_counts:
            pltpu.sync_copy(cnt_sh.at[pl.ds(r0, RPP)], cbuf_v)
            pltpu.sync_copy(cbuf_v, cnt_out.at[c, pl.ds(lo + r0, RPP)])


_agg_with_counts = functools.partial(
    pl.kernel,
    mesh=_mesh,
    out_type=[
        jax.ShapeDtypeStruct((NC, N1, F), jnp.float32),
        jax.ShapeDtypeStruct((NC, N1, CNT_W), jnp.float32),
    ],
    scratch_types=(
        [pltpu.VMEM((CH,), jnp.int32)] * 16
        + [pltpu.VMEM((CH, F), jnp.float32)] * 4
        + [
            pltpu.VMEM((CH, CNT_W), jnp.float32),
            pltpu.VMEM((RPP, CNT_W), jnp.float32),
            pltpu.VMEM_SHARED((NP, F), jnp.float32),
            pltpu.VMEM_SHARED((NP, CNT_W), jnp.float32),
        ]
        + [pltpu.SemaphoreType.DMA] * 12
    ),
)(functools.partial(_agg_body, True))

_agg_no_counts = functools.partial(
    pl.kernel,
    mesh=_mesh,
    out_type=jax.ShapeDtypeStruct((NC, N1, F), jnp.float32),
    scratch_types=(
        [pltpu.VMEM((CH,), jnp.int32)] * 16
        + [pltpu.VMEM((CH, F), jnp.float32)] * 4
        + [pltpu.VMEM_SHARED((NP, F), jnp.float32)]
        + [pltpu.SemaphoreType.DMA] * 12
    ),
)(functools.partial(_agg_body, False))


BLK = 2000


def _sage1_body(sump_ref, cntp_ref, x_ref, w1l_ref, b1_ref, w1r_ref, h1_ref):
    s = sump_ref[0] + sump_ref[1]
    cnt = cntp_ref[0, :, 0:1] + cntp_ref[1, :, 0:1]
    mean = s / jnp.maximum(cnt, 1.0)
    h1_ref[...] = (
        jnp.dot(mean, w1l_ref[...], preferred_element_type=jnp.float32)
        + b1_ref[...]
        + jnp.dot(x_ref[...], w1r_ref[...], preferred_element_type=jnp.float32)
    )


def _sage2_fc_body(sump_ref, cntp_ref, h1_ref, w2l_ref, b2_ref, w2r_ref,
                   fcw1_ref, fcb1_ref, fcw2_ref, fcb2_ref, out_ref):
    s = sump_ref[0] + sump_ref[1]
    cnt = cntp_ref[0, :, 0:1] + cntp_ref[1, :, 0:1]
    mean = s / jnp.maximum(cnt, 1.0)
    h2 = (
        jnp.dot(mean, w2l_ref[...], preferred_element_type=jnp.float32)
        + b2_ref[...]
        + jnp.dot(h1_ref[...], w2r_ref[...], preferred_element_type=jnp.float32)
    )
    t = jnp.dot(h2, fcw1_ref[...], preferred_element_type=jnp.float32) + fcb1_ref[...]
    out_ref[...] = (
        jnp.dot(t, fcw2_ref[...], preferred_element_type=jnp.float32) + fcb2_ref[...]
    )


def _full(shape):
    return pl.BlockSpec(shape, lambda i: tuple(0 for _ in shape))


def _rows(shape):
    return pl.BlockSpec(shape, lambda i: (i,) + tuple(0 for _ in shape[1:]))


def _rows3(shape):
    return pl.BlockSpec(shape, lambda i: (0, i, 0))


def kernel(x, edge_idx, edge_weight, W1l, b1, W1r, W2l, b2, W2r,
           fcW1, fcb1, fcW2, fcb2):
    src = edge_idx[0]
    dst = edge_idx[1]
    pad = E_PAD - E
    src_p = jnp.concatenate([src, jnp.zeros((pad,), jnp.int32)])
    dst_p = jnp.concatenate([dst, jnp.full((pad,), N, jnp.int32)])
    zrow = jnp.zeros((CH, F), jnp.float32)
    zcnt = jnp.zeros((RPP, CNT_W), jnp.float32)
    ones = jnp.ones((CH, CNT_W), jnp.float32)

    sum1_p, cnt_p = _agg_with_counts(x, src_p, dst_p, zrow, zcnt, ones)

    h1 = pl.pallas_call(
        _sage1_body,
        grid=(N // BLK,),
        in_specs=[
            _rows3((NC, BLK, F)),
            _rows3((NC, BLK, CNT_W)),
            _rows((BLK, F)),
            _full((F, H)),
            _full((1, H)),
            _full((F, H)),
        ],
        out_specs=_rows((BLK, H)),
        out_shape=jax.ShapeDtypeStruct((N, H), jnp.float32),
    )(sum1_p, cnt_p, x, W1l, b1.reshape(1, H), W1r)

    sum2_p = _agg_no_counts(h1, src_p, dst_p, zrow, zcnt, ones)

    out = pl.pallas_call(
        _sage2_fc_body,
        grid=(N // BLK,),
        in_specs=[
            _rows3((NC, BLK, H)),
            _rows3((NC, BLK, CNT_W)),
            _rows((BLK, H)),
            _full((H, H)),
            _full((1, H)),
            _full((H, H)),
            _full((H, C)),
            _full((1, C)),
            _full((C, 1)),
            _full((1, 1)),
        ],
        out_specs=_rows((BLK, 1)),
        out_shape=jax.ShapeDtypeStruct((N, 1), jnp.float32),
    )(sum2_p, cnt_p, h1, W2l, b2.reshape(1, H), W2r,
      fcW1, fcb1.reshape(1, C), fcW2, fcb2.reshape(1, 1))

    return out


# speedup vs baseline: 1.1646x; 1.0027x over previous
import functools

import jax
import jax.numpy as jnp
from jax import lax
from jax.experimental import pallas as pl
from jax.experimental.pallas import tpu as pltpu
from jax.experimental.pallas import tpu_sc as plsc

N = 10000
E = 320000
F = 128
H = 128
C = 64

NC = 2
NS = 16
NW = NC * NS
L = 16

CH = 64
NSLOT = 4
P = 7
N1 = 10752
NP = N1 // P
RPP = NP // NS
NCH0 = 240
NCH1 = 80
E_PAD = NS * (NCH0 + NCH1) * CH
CNT_W = 16
IGN = -1

_mesh = plsc.VectorSubcoreMesh(core_axis_name="c", subcore_axis_name="s")


def _agg_body(with_counts, x_hbm, src_hbm, dst_hbm, zrow_hbm, zcnt_hbm,
              ones_hbm, *refs):
    if with_counts:
        sum_out, cnt_out = refs[0], refs[1]
        refs = refs[2:]
    else:
        sum_out = refs[0]
        refs = refs[1:]
    srcs, dsts = refs[0:4], refs[4:8]
    msrcs, mdsts = refs[8:12], refs[12:16]
    rowss = refs[16:20]
    refs = refs[20:]
    if with_counts:
        ones_v, cbuf_v, sum_sh, cnt_sh = refs[0:4]
        sems = refs[4:]
    else:
        sum_sh = refs[0]
        sems = refs[1:]
    gsems, ssems, isems = sems[0:4], sems[4:8], sems[8:12]
    c = lax.axis_index("c")
    s = lax.axis_index("s")
    t0 = NCH0 * CH
    t1 = NCH1 * CH
    e0 = jnp.where(c == 0, s * t0, NS * t0 + s * t1)
    nch = jnp.where(c == 0, NCH0, NCH1)
    r0 = s * RPP

    if with_counts:
        pltpu.sync_copy(ones_hbm, ones_v)

    def gather_desc(u):
        return pltpu.make_async_copy(
            x_hbm.at[plsc.Indices(msrcs[u], ignored_value=IGN)],
            rowss[u], gsems[u])

    def scat_desc(u):
        return pltpu.make_async_copy(
            rowss[u], sum_sh.at[plsc.Indices(mdsts[u], ignored_value=IGN)],
            ssems[u])

    def cnt_desc(u):
        return pltpu.make_async_copy(
            ones_v, cnt_sh.at[plsc.Indices(mdsts[u], ignored_value=IGN)],
            ssems[u])

    for p in range(P):
        lo = p * NP

        pltpu.sync_copy(zrow_hbm, rowss[0])
        pltpu.sync_copy(rowss[0], sum_sh.at[pl.ds(r0, CH)])
        pltpu.sync_copy(rowss[0].at[pl.ds(0, RPP - CH)],
                        sum_sh.at[pl.ds(r0 + CH, RPP - CH)])
        if with_counts:
            pltpu.sync_copy(zcnt_hbm, cbuf_v)
            pltpu.sync_copy(cbuf_v, cnt_sh.at[pl.ds(r0, RPP)])
        plsc.subcore_barrier()

        def iload(cj, u):
            off = e0 + cj * CH
            pltpu.async_copy(src_hbm.at[pl.ds(off, CH)], srcs[u], isems[u])
            pltpu.async_copy(dst_hbm.at[pl.ds(off, CH)], dsts[u], isems[u])

        def iwait(cj, u):
            off = e0 + cj * CH
            pltpu.make_async_copy(
                src_hbm.at[pl.ds(off, CH)], srcs[u], isems[u]).wait()
            pltpu.make_async_copy(
                dst_hbm.at[pl.ds(off, CH)], dsts[u], isems[u]).wait()

        def mask_gather(cj, u):
            for k in range(CH // L):
                sl = pl.ds(k * L, L)
                d = dsts[u][sl]
                sv = srcs[u][sl]
                inr = (d >= lo) & (d < lo + NP)
                mdsts[u][sl] = jnp.where(inr, d - lo, IGN)
                msrcs[u][sl] = jnp.where(inr, sv, IGN)
            gather_desc(u).start()

        iload(0, 0)
        iload(1, 1)
        iwait(0, 0)
        mask_gather(0, 0)
        iwait(1, 1)
        mask_gather(1, 1)
        iload(2, 2)
        iload(3, 3)

        def body(jj, carry):
            for u in range(NSLOT):
                cj = NSLOT * jj + u
                u2 = (u + 2) % NSLOT

                @pl.when(cj >= 2)
                def _():
                    scat_desc(u2).wait()
                    if with_counts:
                        cnt_desc(u2).wait()

                @pl.when(cj + 2 < nch)
                def _():
                    iwait(cj + 2, u2)
                    mask_gather(cj + 2, u2)

                @pl.when(cj + 4 < nch)
                def _():
                    iload(cj + 4, u)

                gather_desc(u).wait()
                scat_desc(u).start(add=True)
                if with_counts:
                    cnt_desc(u).start(add=True)
            return carry

        lax.fori_loop(0, nch // NSLOT, body, 0)
        for u in (2, 3):
            scat_desc(u).wait()
            if with_counts:
                cnt_desc(u).wait()

        plsc.subcore_barrier()
        pltpu.sync_copy(sum_sh.at[pl.ds(r0, CH)], rowss[0])
        pltpu.sync_copy(rowss[0], sum_out.at[c, pl.ds(lo + r0, CH)])
        pltpu.sync_copy(sum_sh.at[pl.ds(r0 + CH, RPP - CH)],
                        rowss[1].at[pl.ds(0, RPP - CH)])
        pltpu.sync_copy(rowss[1].at[pl.ds(0, RPP - CH)],
                        sum_out.at[c, pl.ds(lo + r0 + CH, RPP - CH)])
        if with_counts:
            pltpu.sync_copy(cnt_sh.at[pl.ds(r0, RPP)], cbuf_v)
            pltpu.sync_copy(cbuf_v, cnt_out.at[c, pl.ds(lo + r0, RPP)])


_agg_with_counts = functools.partial(
    pl.kernel,
    mesh=_mesh,
    out_type=[
        jax.ShapeDtypeStruct((NC, N1, F), jnp.float32),
        jax.ShapeDtypeStruct((NC, N1, CNT_W), jnp.float32),
    ],
    scratch_types=(
        [pltpu.VMEM((CH,), jnp.int32)] * 16
        + [pltpu.VMEM((CH, F), jnp.float32)] * 4
        + [
            pltpu.VMEM((CH, CNT_W), jnp.float32),
            pltpu.VMEM((RPP, CNT_W), jnp.float32),
            pltpu.VMEM_SHARED((NP, F), jnp.float32),
            pltpu.VMEM_SHARED((NP, CNT_W), jnp.float32),
        ]
        + [pltpu.SemaphoreType.DMA] * 12
    ),
)(functools.partial(_agg_body, True))

_agg_no_counts = functools.partial(
    pl.kernel,
    mesh=_mesh,
    out_type=jax.ShapeDtypeStruct((NC, N1, F), jnp.float32),
    scratch_types=(
        [pltpu.VMEM((CH,), jnp.int32)] * 16
        + [pltpu.VMEM((CH, F), jnp.float32)] * 4
        + [pltpu.VMEM_SHARED((NP, F), jnp.float32)]
        + [pltpu.SemaphoreType.DMA] * 12
    ),
)(functools.partial(_agg_body, False))


BLK = 2000


def _sage1_body(sump_ref, cntp_ref, x_ref, w1l_ref, b1_ref, w1r_ref, h1_ref):
    s = sump_ref[0] + sump_ref[1]
    cnt = cntp_ref[0, :, 0:1] + cntp_ref[1, :, 0:1]
    mean = s / jnp.maximum(cnt, 1.0)
    h1_ref[...] = (
        jnp.dot(mean, w1l_ref[...], preferred_element_type=jnp.float32)
        + b1_ref[...]
        + jnp.dot(x_ref[...], w1r_ref[...], preferred_element_type=jnp.float32)
    )


def _sage2_fc_body(sump_ref, cntp_ref, h1_ref, w2l_ref, b2_ref, w2r_ref,
                   fcw1_ref, fcb1_ref, fcw2_ref, fcb2_ref, out_ref):
    s = sump_ref[0] + sump_ref[1]
    cnt = cntp_ref[0, :, 0:1] + cntp_ref[1, :, 0:1]
    mean = s / jnp.maximum(cnt, 1.0)
    h2 = (
        jnp.dot(mean, w2l_ref[...], preferred_element_type=jnp.float32)
        + b2_ref[...]
        + jnp.dot(h1_ref[...], w2r_ref[...], preferred_element_type=jnp.float32)
    )
    t = jnp.dot(h2, fcw1_ref[...], preferred_element_type=jnp.float32) + fcb1_ref[...]
    out_ref[...] = (
        jnp.dot(t, fcw2_ref[...], preferred_element_type=jnp.float32) + fcb2_ref[...]
    )


def _full(shape):
    return pl.BlockSpec(shape, lambda i: tuple(0 for _ in shape))


def _rows(shape):
    return pl.BlockSpec(shape, lambda i: (i,) + tuple(0 for _ in shape[1:]))


def _rows3(shape):
    return pl.BlockSpec(shape, lambda i: (0, i, 0))


def kernel(x, edge_idx, edge_weight, W1l, b1, W1r, W2l, b2, W2r,
           fcW1, fcb1, fcW2, fcb2):
    src = edge_idx[0]
    dst = edge_idx[1]
    pad = E_PAD - E
    src_p = jnp.concatenate([src, jnp.zeros((pad,), jnp.int32)])
    dst_p = jnp.concatenate([dst, jnp.full((pad,), N, jnp.int32)])
    zrow = jnp.zeros((CH, F), jnp.float32)
    zcnt = jnp.zeros((RPP, CNT_W), jnp.float32)
    ones = jnp.ones((CH, CNT_W), jnp.float32)

    sum1_p, cnt_p = _agg_with_counts(x, src_p, dst_p, zrow, zcnt, ones)

    h1 = pl.pallas_call(
        _sage1_body,
        grid=(N // BLK,),
        in_specs=[
            _rows3((NC, BLK, F)),
            _rows3((NC, BLK, CNT_W)),
            _rows((BLK, F)),
            _full((F, H)),
            _full((1, H)),
            _full((F, H)),
        ],
        out_specs=_rows((BLK, H)),
        out_shape=jax.ShapeDtypeStruct((N, H), jnp.float32),
    )(sum1_p, cnt_p, x, W1l, b1.reshape(1, H), W1r)

    sum2_p = _agg_no_counts(h1, src_p, dst_p, zrow, zcnt, ones)

    out = pl.pallas_call(
        _sage2_fc_body,
        grid=(N // BLK,),
        in_specs=[
            _rows3((NC, BLK, H)),
            _rows3((NC, BLK, CNT_W)),
            _rows((BLK, H)),
            _full((H, H)),
            _full((1, H)),
            _full((H, H)),
            _full((H, C)),
            _full((1, C)),
            _full((C, 1)),
            _full((1, 1)),
        ],
        out_specs=_rows((BLK, 1)),
        out_shape=jax.ShapeDtypeStruct((N, 1), jnp.float32),
    )(sum2_p, cnt_p, h1, W2l, b2.reshape(1, H), W2r,
      fcW1, fcb1.reshape(1, C), fcW2, fcb2.reshape(1, 1))

    return out
